# Initial kernel scaffold; baseline (speedup 1.0000x reference)
import jax
import jax.numpy as jnp
from jax.experimental import pallas as pl


def kernel(patch_logits, coords, prev_pred):
    raise NotImplementedError("write your pallas kernel here")


# speedup vs baseline: 57.4023x; 57.4023x over previous
import functools

import jax
import jax.numpy as jnp
from jax import lax
from jax.experimental import pallas as pl
from jax.experimental.pallas import tpu as pltpu
from jax.experimental.pallas import tpu_sc as plsc

B, K, PS, H, W = 16, 256, 64, 512, 512
NW = 32
BAND = H // NW
RCH = 16
L = 16
SLAB = RCH * PS


def _agg_body(patches_hbm, coords_hbm, prev_hbm, out_hbm,
              coords_v, stage_v, out_v, cnt_v, prev_v,
              src_s, d0_s, w_s, sem0, sem1, psem):
    wid = lax.axis_index("s") * 2 + lax.axis_index("c")

    zeros = jnp.zeros((L,), jnp.float32)
    ones = jnp.ones((L,), jnp.float32)

    def per_batch(b, carry):
        band = lax.rem(wid + b, NW)
        s_row = band * BAND

        pltpu.sync_copy(coords_hbm.at[pl.ds(b * (2 * K), 2 * K)],
                        coords_v.at[pl.ds(0, 2 * K)])
        prev_cp = pltpu.async_copy(
            prev_hbm.at[pl.ds((b * H + s_row) * W, BAND * W)], prev_v, psem)

        def zbody(m, c):
            out_v[pl.ds(m * L, L)] = zeros
            cnt_v[pl.ds(m * L, L)] = zeros
            return c
        lax.fori_loop(0, BAND * W // L, zbody, 0)

        def scan(k, n):
            cvec = coords_v[pl.ds(2 * k, L)]
            h = cvec[0]
            w = cvec[1]
            valid = (h < s_row + BAND) & (h + PS > s_row)

            @pl.when(valid)
            def _():
                p0 = jnp.minimum(jnp.maximum(s_row - h, 0), PS - RCH)
                src_s[n] = ((b * K + k) * PS + p0) * PS
                d0_s[n] = h + p0 - s_row
                w_s[n] = w
            return n + valid.astype(jnp.int32)
        n = lax.fori_loop(0, K, scan, 0)

        @pl.when(n > 0)
        def _():
            src0 = pl.multiple_of(src_s[0], PS)
            pltpu.async_copy(patches_hbm.at[pl.ds(src0, SLAB)],
                             stage_v.at[pl.ds(0, SLAB)], sem0)

        def pipe(i, c):
            par = lax.rem(i, 2)
            pbase = par * SLAB

            @pl.when(par == 0)
            def _():
                pltpu.make_async_copy(
                    patches_hbm.at[pl.ds(0, SLAB)],
                    stage_v.at[pl.ds(0, SLAB)], sem0).wait()

            @pl.when(par == 1)
            def _():
                pltpu.make_async_copy(
                    patches_hbm.at[pl.ds(0, SLAB)],
                    stage_v.at[pl.ds(SLAB, SLAB)], sem1).wait()

            @pl.when(i + 1 < n)
            def _():
                nsrc = pl.multiple_of(src_s[i + 1], PS)

                @pl.when(par == 0)
                def _():
                    pltpu.async_copy(patches_hbm.at[pl.ds(nsrc, SLAB)],
                                     stage_v.at[pl.ds(SLAB, SLAB)], sem1)

                @pl.when(par == 1)
                def _():
                    pltpu.async_copy(patches_hbm.at[pl.ds(nsrc, SLAB)],
                                     stage_v.at[pl.ds(0, SLAB)], sem0)

            d0 = d0_s[i]
            w = w_s[i]
            for j in range(RCH):
                d = d0 + j

                @pl.when((d >= 0) & (d < BAND))
                def _():
                    base = d * W + w
                    for cc in range(PS // L):
                        sl = pl.ds(base + cc * L, L)
                        plsc.addupdate(
                            out_v.at[sl],
                            stage_v[pl.ds(pbase + j * PS + cc * L, L)])
                        plsc.addupdate(cnt_v.at[sl], ones)
            return c
        lax.fori_loop(0, n, pipe, 0, unroll=False)

        prev_cp.wait()

        def fbody(m, c):
            sl = pl.ds(m * L, L)
            out_v[sl] = (out_v[sl] / jnp.maximum(cnt_v[sl], ones)
                         + prev_v[sl]) * 0.5
            return c
        lax.fori_loop(0, BAND * W // L, fbody, 0)

        pltpu.sync_copy(out_v, out_hbm.at[pl.ds((b * H + s_row) * W, BAND * W)])
        return carry

    lax.fori_loop(0, B, per_batch, 0)


_agg = functools.partial(
    pl.kernel,
    out_type=jax.ShapeDtypeStruct((B * H * W,), jnp.float32),
    mesh=plsc.VectorSubcoreMesh(core_axis_name="c", subcore_axis_name="s"),
    scratch_types=[
        pltpu.VMEM((2 * K + L,), jnp.int32),
        pltpu.VMEM((2 * SLAB,), jnp.float32),
        pltpu.VMEM((BAND * W,), jnp.float32),
        pltpu.VMEM((BAND * W,), jnp.float32),
        pltpu.VMEM((BAND * W,), jnp.float32),
        pltpu.SMEM((K,), jnp.int32),
        pltpu.SMEM((K,), jnp.int32),
        pltpu.SMEM((K,), jnp.int32),
        pltpu.SemaphoreType.DMA,
        pltpu.SemaphoreType.DMA,
        pltpu.SemaphoreType.DMA,
    ],
)(_agg_body)


def kernel(patch_logits, coords, prev_pred):
    patches = patch_logits.reshape(-1)
    coords_f = coords.reshape(-1)
    prev = prev_pred.reshape(-1)
    out = _agg(patches, coords_f, prev)
    return out.reshape(B, 1, H, W)
